# baseline (speedup 1.0000x reference)
import functools

import jax
import jax.numpy as jnp
from jax import lax
from jax.experimental import pallas as pl
from jax.experimental.pallas import tpu as pltpu
from jax.experimental.pallas import tpu_sc as plsc

NC = 2
NS = 16
NW = NC * NS
CH = 128
SUP = 2
NB = 3


def _build(B, H, T_kv, T_q, n_sel, D):
    rows_total = B * H * T_q * n_sel
    rows_per_w = rows_total // NW
    group_rows = T_q * n_sel
    groups_per_w = rows_per_w // group_rows
    srows = SUP * CH
    n2 = rows_per_w // srows
    sup_per_group = group_rows // srows

    mesh = plsc.VectorSubcoreMesh(core_axis_name="c", subcore_axis_name="s")

    scratch = ([pltpu.VMEM((srows,), jnp.int32) for _ in range(NB)]
               + [pltpu.VMEM((srows, D), jnp.float32) for _ in range(NB)]
               + [pltpu.SemaphoreType.DMA for _ in range(3 * NB)])

    @functools.partial(
        pl.kernel,
        mesh=mesh,
        out_type=jax.ShapeDtypeStruct((rows_total, D), jnp.float32),
        scratch_types=scratch,
    )
    def gather_kernel(kv_hbm, idx_hbm, out_hbm, *sc):
        idx_bufs = sc[:NB]
        rows_bufs = sc[NB:2 * NB]
        gsems = sc[2 * NB:3 * NB]
        ssems = sc[3 * NB:4 * NB]
        isems = sc[4 * NB:5 * NB]

        wid = lax.axis_index("s") * NC + lax.axis_index("c")
        w_row0 = wid * rows_per_w

        def row0_of(s):
            return w_row0 + s * srows

        def idx_load(s, b):
            pltpu.async_copy(idx_hbm.at[pl.ds(row0_of(s), srows)],
                             idx_bufs[b], isems[b])

        def idx_wait(b):
            pltpu.make_async_copy(idx_hbm.at[pl.ds(w_row0, srows)],
                                  idx_bufs[b], isems[b]).wait()

        def gather2(s, b):
            base = ((wid * groups_per_w + s // sup_per_group) * T_kv)
            tab = kv_hbm.at[pl.ds(base, T_kv)]
            for h in range(SUP):
                pltpu.async_copy(tab.at[idx_bufs[b].at[pl.ds(h * CH, CH)]],
                                 rows_bufs[b].at[pl.ds(h * CH, CH)],
                                 gsems[b])

        def gather2_wait(b):
            for h in range(SUP):
                pltpu.make_async_copy(
                    kv_hbm.at[pl.ds(0, T_kv)]
                          .at[idx_bufs[b].at[pl.ds(h * CH, CH)]],
                    rows_bufs[b].at[pl.ds(h * CH, CH)],
                    gsems[b]).wait()

        def store(s, b):
            pltpu.async_copy(rows_bufs[b],
                             out_hbm.at[pl.ds(row0_of(s), srows)], ssems[b])

        def store_wait(b):
            pltpu.make_async_copy(rows_bufs[b],
                                  out_hbm.at[pl.ds(w_row0, srows)],
                                  ssems[b]).wait()

        def retire(s, bl):
            gather2_wait(bl)
            store(s - 1, bl)
            idx_load(jnp.minimum(s + 2, n2 - 1), bl)

        for b in range(NB):
            idx_load(b, b)
        for s in range(NB):
            b = s % NB
            idx_wait(b)
            gather2(s, b)
            if s >= 1:
                retire(s, (s - 1) % NB)

        def body(s, b):
            idx_wait(b)
            store_wait(b)
            gather2(s, b)
            retire(s, (b - 1) % NB)

        def blk(q, carry):
            s0 = NB * q + NB
            for i in range(NB):
                body(s0 + i, i)
            return carry

        n_blk = (n2 - NB) // NB
        lax.fori_loop(0, n_blk, blk, 0)
        for s in range(NB * n_blk + NB, n2):
            body(s, s % NB)

        last_b = (n2 - 1) % NB
        gather2_wait(last_b)
        store(n2 - 1, last_b)
        for s in range(n2 - 2, n2):
            idx_wait((s + 2) % NB)
        for b in range(NB):
            store_wait(b)

    return gather_kernel


def kernel(kv_states, indices):
    B, H, T_kv, D = kv_states.shape
    _, _, T_q, n_sel = indices.shape
    kv_flat = kv_states.reshape(B * H * T_kv, D)
    idx_flat = indices.reshape(-1).astype(jnp.int32)
    out = _build(B, H, T_kv, T_q, n_sel, D)(kv_flat, idx_flat)
    return out.reshape(B, H, T_q, n_sel, D)


# speedup vs baseline: 1.0121x; 1.0121x over previous
import functools

import jax
import jax.numpy as jnp
from jax import lax
from jax.experimental import pallas as pl
from jax.experimental.pallas import tpu as pltpu
from jax.experimental.pallas import tpu_sc as plsc

NC = 2
NS = 16
NW = NC * NS
LANES = 16
CH = 128
NBUF = 4
L = 2


def _build(B, H, T_kv, T_q, n_sel, D):
    rows_total = B * H * T_q * n_sel
    rows_per_w = rows_total // NW
    group_rows = T_q * n_sel
    groups_per_w = rows_per_w // group_rows
    n = rows_per_w // CH
    chunks_per_group = group_rows // CH
    assert n % NBUF == 0 and NBUF > L

    mesh = plsc.VectorSubcoreMesh(core_axis_name="c", subcore_axis_name="s")

    scratch = ([pltpu.VMEM((CH,), jnp.int32) for _ in range(NBUF)]
               + [pltpu.VMEM((CH, D), jnp.float32) for _ in range(NBUF)]
               + [pltpu.SemaphoreType.DMA for _ in range(3 * NBUF)])

    @functools.partial(
        pl.kernel,
        mesh=mesh,
        out_type=jax.ShapeDtypeStruct((rows_total, D), jnp.float32),
        scratch_types=scratch,
    )
    def gather_kernel(kv_hbm, idx_hbm, out_hbm, *sc):
        idx_bufs = sc[:NBUF]
        rows_bufs = sc[NBUF:2 * NBUF]
        gsems = sc[2 * NBUF:3 * NBUF]
        ssems = sc[3 * NBUF:4 * NBUF]
        isems = sc[4 * NBUF:5 * NBUF]

        wid = lax.axis_index("s") * NC + lax.axis_index("c")
        w_row0 = wid * rows_per_w

        def row0_of(j):
            return w_row0 + j * CH

        def idx_load(j, b):
            pltpu.async_copy(idx_hbm.at[pl.ds(row0_of(j), CH)],
                             idx_bufs[b], isems[b])

        def idx_wait(b):
            pltpu.make_async_copy(idx_hbm.at[pl.ds(w_row0, CH)],
                                  idx_bufs[b], isems[b]).wait()

        def rebase(j, b):
            base = ((wid * groups_per_w + j // chunks_per_group) * T_kv)
            bvec = jnp.broadcast_to(jnp.int32(0) + base, (LANES,))
            ref = idx_bufs[b]
            for k in range(CH // LANES):
                sl = pl.ds(LANES * k, LANES)
                ref[sl] = ref[sl] + bvec

        def gather(j, b):
            base = ((wid * groups_per_w + j // chunks_per_group) * T_kv)
            pltpu.async_copy(kv_hbm.at[pl.ds(base, T_kv)].at[idx_bufs[b]],
                             rows_bufs[b], gsems[b])

        def gather_wait(b):
            pltpu.make_async_copy(kv_hbm.at[pl.ds(0, T_kv)].at[idx_bufs[b]],
                                  rows_bufs[b], gsems[b]).wait()

        def store(j, b):
            pltpu.async_copy(rows_bufs[b],
                             out_hbm.at[pl.ds(row0_of(j), CH)], ssems[b])

        def store_wait(b):
            pltpu.make_async_copy(rows_bufs[b],
                                  out_hbm.at[pl.ds(w_row0, CH)],
                                  ssems[b]).wait()

        def retire(j, bl):
            gather_wait(bl)
            store(j - L, bl)
            idx_load(jnp.minimum(j - L + NBUF, n - 1), bl)

        for b in range(NBUF):
            idx_load(b, b)
        for j in range(NBUF):
            b = j % NBUF
            idx_wait(b)
            gather(j, b)
            if j >= L:
                retire(j, (j - L) % NBUF)

        def body(j, b):
            idx_wait(b)
            store_wait(b)
            gather(j, b)
            retire(j, (b - L) % NBUF)

        def blk(q, carry):
            j0 = NBUF * q + NBUF
            for i in range(NBUF):
                body(j0 + i, i)
            return carry

        lax.fori_loop(0, (n - NBUF) // NBUF, blk, 0)

        for t in range(L):
            jj = n - L + t
            b = jj % NBUF
            gather_wait(b)
            store(jj, b)
        for t in range(L):
            idx_wait((n + NBUF - 2 * L + t) % NBUF)
        for b in range(NBUF):
            store_wait(b)

    return gather_kernel


def kernel(kv_states, indices):
    B, H, T_kv, D = kv_states.shape
    _, _, T_q, n_sel = indices.shape
    kv_flat = kv_states.reshape(B * H * T_kv, D)
    idx_flat = indices.reshape(-1).astype(jnp.int32)
    out = _build(B, H, T_kv, T_q, n_sel, D)(kv_flat, idx_flat)
    return out.reshape(B, H, T_q, n_sel, D)
